# baseline (speedup 1.0000x reference)
import functools

import jax
import jax.numpy as jnp
from jax import lax
from jax.experimental import pallas as pl
from jax.experimental.pallas import tpu as pltpu
from jax.experimental.pallas import tpu_sc as plsc

NS = 16
LANES = 16
BATCH = 16


def _ceil_to(x: int, m: int) -> int:
    return (x + m - 1) // m * m


def _rsqrt16(d):
    i = lax.bitcast_convert_type(d, jnp.int32)
    y = lax.bitcast_convert_type(jnp.int32(0x5F3759DF) - (i >> 1),
                                 jnp.float32)
    half_d = 0.5 * d
    for _ in range(3):
        y = y * (1.5 - half_d * y * y)
    return y


def _gather_scatter(table_sh, acc_sh, src_v, dst_v, vals_v,
                    semg0, semg1, sems, rows):
    semg = (semg0, semg1)
    bounds = list(range(0, rows, BATCH)) + [rows]
    batches = [(bounds[k], bounds[k + 1]) for k in range(len(bounds) - 1)
               if bounds[k] < bounds[k + 1]]
    nb = len(batches)

    def fire_gathers(k):
        lo, hi = batches[k]
        return [pltpu.async_copy(table_sh.at[src_v.at[j]], vals_v.at[j],
                                 semg[k % 2])
                for j in range(lo, hi)]

    inflight = {0: fire_gathers(0)}
    if nb > 1:
        inflight[1] = fire_gathers(1)
    scats = []
    for k in range(nb):
        for d in inflight.pop(k):
            d.wait()
        lo, hi = batches[k]
        scats += [pltpu.async_copy(vals_v.at[j], acc_sh.at[dst_v.at[j]],
                                   sems, add=True)
                  for j in range(lo, hi)]
        if k + 2 < nb:
            inflight[k + 2] = fire_gathers(k + 2)
    for d in scats:
        d.wait()


@functools.cache
def _sc_pass_a(npad: int, rows: int, row_w: int):
    zch = npad // NS

    def body(ei_hbm, x_hbm, s_hbm, dinv_hbm,
             src_v, dst_v, vals_v, ones_v, zrow_v, xb_v, dinv_v, u_v,
             table_sh, acc_sh, sem, semg0, semg1, sems):
        s = lax.axis_index("s")
        sl = pl.ds(s * zch, zch)
        base = pl.multiple_of(rows * s, 8)
        es_d = pltpu.async_copy(ei_hbm.at[0, pl.ds(base, rows)], src_v, sem)
        ed_d = pltpu.async_copy(ei_hbm.at[1, pl.ds(base, rows)], dst_v, sem)
        ix_d = pltpu.async_copy(x_hbm.at[sl], xb_v, sem)
        zero16 = jnp.zeros((LANES,), jnp.float32)
        one16 = jnp.ones((LANES,), jnp.float32)
        for i in range(zch // LANES):
            zrow_v[pl.ds(i * LANES, LANES)] = zero16
        for i in range(row_w // LANES + 1):
            ones_v[pl.ds(i * LANES, LANES)] = one16
        pltpu.sync_copy(zrow_v, acc_sh.at[sl])
        es_d.wait()
        ed_d.wait()
        ix_d.wait()
        plsc.subcore_barrier()

        scat = [pltpu.async_copy(ones_v.at[pl.ds(0, row_w)],
                                 acc_sh.at[dst_v.at[j]], sems, add=True)
                for j in range(rows)]
        for d in scat:
            d.wait()
        plsc.subcore_barrier()

        pltpu.sync_copy(acc_sh.at[sl], u_v)
        for i in range(zch // LANES):
            ii = pl.ds(i * LANES, LANES)
            dinv = _rsqrt16(u_v[ii] + 1.0)
            dinv_v[ii] = dinv
            u_v[ii] = dinv * xb_v[ii]
        pltpu.sync_copy(zrow_v, acc_sh.at[sl])
        pltpu.sync_copy(u_v, table_sh.at[sl])
        pltpu.sync_copy(dinv_v, dinv_hbm.at[sl])
        plsc.subcore_barrier()

        _gather_scatter(table_sh, acc_sh, src_v, dst_v, vals_v,
                        semg0, semg1, sems, rows)
        plsc.subcore_barrier()

        pltpu.sync_copy(acc_sh.at[sl], xb_v)
        for i in range(zch // LANES):
            ii = pl.ds(i * LANES, LANES)
            xb_v[ii] = dinv_v[ii] * (xb_v[ii] + u_v[ii])
        pltpu.sync_copy(xb_v, s_hbm.at[sl])

    return pl.kernel(
        body,
        out_type=(
            jax.ShapeDtypeStruct((npad,), jnp.float32),
            jax.ShapeDtypeStruct((npad,), jnp.float32),
        ),
        mesh=plsc.VectorSubcoreMesh(core_axis_name="c", subcore_axis_name="s",
                                    num_cores=1),
        scratch_types=[
            pltpu.VMEM((rows, row_w), jnp.int32),
            pltpu.VMEM((rows, row_w), jnp.int32),
            pltpu.VMEM((rows, row_w), jnp.float32),
            pltpu.VMEM((_ceil_to(row_w + 1, LANES),), jnp.float32),
            pltpu.VMEM((zch,), jnp.float32),
            pltpu.VMEM((zch,), jnp.float32),
            pltpu.VMEM((zch,), jnp.float32),
            pltpu.VMEM((zch,), jnp.float32),
            pltpu.VMEM_SHARED((npad,), jnp.float32),
            pltpu.VMEM_SHARED((npad,), jnp.float32),
            pltpu.SemaphoreType.DMA,
            pltpu.SemaphoreType.DMA,
            pltpu.SemaphoreType.DMA,
            pltpu.SemaphoreType.DMA,
        ],
    )


@functools.cache
def _sc_pass_b(npad: int, rows: int, row_w: int):
    zch = npad // NS

    def body(ei_hbm, table_hbm, dinv_hbm, b2_hbm, out_hbm,
             src_v, dst_v, vals_v, zrow_v, tab_v, dinv_v, b2_v,
             table_sh, acc_sh, sem, semg0, semg1, sems):
        s = lax.axis_index("s")
        sl = pl.ds(s * zch, zch)
        base = pl.multiple_of(rows * s, 8)
        es_d = pltpu.async_copy(ei_hbm.at[0, pl.ds(base, rows)], src_v, sem)
        ed_d = pltpu.async_copy(ei_hbm.at[1, pl.ds(base, rows)], dst_v, sem)
        tab_d = pltpu.async_copy(table_hbm.at[sl], tab_v, sem)
        idv_d = pltpu.async_copy(dinv_hbm.at[sl], dinv_v, sem)
        ib2_d = pltpu.async_copy(b2_hbm, b2_v, sem)
        zero16 = jnp.zeros((LANES,), jnp.float32)
        for i in range(zch // LANES):
            zrow_v[pl.ds(i * LANES, LANES)] = zero16
        pltpu.sync_copy(zrow_v, acc_sh.at[sl])
        tab_d.wait()
        idv_d.wait()
        ib2_d.wait()
        es_d.wait()
        ed_d.wait()
        pltpu.sync_copy(tab_v, table_sh.at[sl])
        plsc.subcore_barrier()

        _gather_scatter(table_sh, acc_sh, src_v, dst_v, vals_v,
                        semg0, semg1, sems, rows)
        plsc.subcore_barrier()

        pltpu.sync_copy(acc_sh.at[sl], zrow_v)
        b2 = b2_v[pl.ds(0, LANES)]
        for i in range(zch // LANES):
            ii = pl.ds(i * LANES, LANES)
            zrow_v[ii] = dinv_v[ii] * (zrow_v[ii] + tab_v[ii]) + b2
        pltpu.sync_copy(zrow_v, out_hbm.at[sl])

    return pl.kernel(
        body,
        out_type=jax.ShapeDtypeStruct((npad,), jnp.float32),
        mesh=plsc.VectorSubcoreMesh(core_axis_name="c", subcore_axis_name="s",
                                    num_cores=1),
        scratch_types=[
            pltpu.VMEM((rows, row_w), jnp.int32),
            pltpu.VMEM((rows, row_w), jnp.int32),
            pltpu.VMEM((rows, row_w), jnp.float32),
            pltpu.VMEM((zch,), jnp.float32),
            pltpu.VMEM((zch,), jnp.float32),
            pltpu.VMEM((zch,), jnp.float32),
            pltpu.VMEM((LANES,), jnp.float32),
            pltpu.VMEM_SHARED((npad,), jnp.float32),
            pltpu.VMEM_SHARED((npad,), jnp.float32),
            pltpu.SemaphoreType.DMA,
            pltpu.SemaphoreType.DMA,
            pltpu.SemaphoreType.DMA,
            pltpu.SemaphoreType.DMA,
        ],
    )


@functools.cache
def _tc_dense(npad: int, hidden: int, bb: int):
    grid = npad // bb

    def body(srow, dinv, w1c, b1c, w2c, v_out):
        dv = dinv[...]
        sv = srow[...]
        h = jnp.maximum(w1c[...] * sv + b1c[...], 0.0)
        t = jnp.sum(w2c[...] * h, axis=0, keepdims=True)
        v_out[...] = dv * t

    node = pl.BlockSpec((1, bb), lambda i: (0, i))
    wcol = pl.BlockSpec((hidden, 1), lambda i: (0, 0))
    return pl.pallas_call(
        body,
        grid=(grid,),
        in_specs=[node, node, wcol, wcol, wcol],
        out_specs=node,
        out_shape=jax.ShapeDtypeStruct((1, npad), jnp.float32),
    )


def _pick_row_w(e: int) -> tuple[int, int]:
    for row_w in range(128, 0, -1):
        if e % (NS * row_w):
            continue
        rows = e // (NS * row_w)
        if rows % 8 == 0:
            return row_w, rows
    raise NotImplementedError(f"no valid row split for {e} edges")


def kernel(x, edge_index, W1, b1, W2, b2):
    n = x.shape[0]
    e = edge_index.shape[1]
    hidden = W1.shape[1]

    npad = _ceil_to(n, 2048)
    row_w, rows = _pick_row_w(e)

    ei3 = edge_index.astype(jnp.int32).reshape(2, NS * rows, row_w)
    xpad = jnp.zeros((npad,), jnp.float32).at[:n].set(x[:, 0])
    w1c = W1.reshape(hidden, 1)
    b1c = b1.reshape(hidden, 1)
    w2c = W2.reshape(hidden, 1)
    b2v = jnp.full((LANES,), b2[0], jnp.float32)

    s_arr, dinv = _sc_pass_a(npad, rows, row_w)(ei3, xpad)
    v = _tc_dense(npad, hidden, 2048)(s_arr.reshape(1, npad),
                                      dinv.reshape(1, npad),
                                      w1c, b1c, w2c)
    out = _sc_pass_b(npad, rows, row_w)(ei3, v.reshape(npad), dinv, b2v)
    return out.reshape(npad, 1)[:n]
